# Initial kernel scaffold; baseline (speedup 1.0000x reference)
import jax
import jax.numpy as jnp
from jax.experimental import pallas as pl


def kernel(story, hidden, C0, C1, C2, C3):
    raise NotImplementedError("write your pallas kernel here")


# speedup vs baseline: 3.7107x; 3.7107x over previous
import functools

import jax
import jax.numpy as jnp
from jax import lax
from jax.experimental import pallas as pl
from jax.experimental.pallas import tpu as pltpu
from jax.experimental.pallas import tpu_sc as plsc

VOCAB = 100000
D = 128
B, M, S = 1024, 200, 4
POS = B * M
NC, NS = 2, 16
NW = NC * NS
POS_W = POS // NW
GP = 32
ROWS = GP * S
CHUNKS = POS_W // GP


def _sc_gather_pool(flat_idx, C0, C1, C2, C3):
    mesh = plsc.VectorSubcoreMesh(core_axis_name="c", subcore_axis_name="s")
    out = jax.ShapeDtypeStruct((POS, D), jnp.float32)

    @functools.partial(
        pl.kernel,
        mesh=mesh,
        out_type=[out, out, out, out],
        scratch_types=[
            pltpu.VMEM((ROWS,), jnp.int32),
            pltpu.VMEM((ROWS, D), jnp.float32),
            pltpu.VMEM((GP, D), jnp.float32),
            pltpu.SemaphoreType.DMA,
        ],
    )
    def k(idx_hbm, t0, t1, t2, t3, e0, e1, e2, e3, idx_v, raw_v, pool_v, sem):
        wid = lax.axis_index("s") * NC + lax.axis_index("c")
        base = wid * POS_W

        for table, e_out in ((t0, e0), (t1, e1), (t2, e2), (t3, e3)):
            def chunk_body(ci, _, table=table, e_out=e_out):
                pos0 = base + ci * GP
                pltpu.sync_copy(idx_hbm.at[pl.ds(pos0 * S, ROWS)], idx_v)
                pltpu.async_copy(table.at[idx_v], raw_v, sem).wait()

                def pool_body(i, _):
                    r = i * S
                    for j in range(D // 16):
                        sl = pl.ds(j * 16, 16)
                        acc = raw_v[r, sl] + raw_v[r + 1, sl]
                        acc = acc + raw_v[r + 2, sl]
                        pool_v[i, sl] = acc + raw_v[r + 3, sl]
                    return 0

                lax.fori_loop(0, GP, pool_body, 0)
                pltpu.sync_copy(pool_v, e_out.at[pl.ds(pos0, GP)])
                return 0

            lax.fori_loop(0, CHUNKS, chunk_body, 0)

    return k(flat_idx, C0, C1, C2, C3)


BB = 32


def _hops_body(e0, e1, e2, e3, h, lg_out, u_out):
    u = h[:, 0, :]
    tables = (e0, e1, e2, e3)
    logit = None
    for hop in range(3):
        eh = tables[hop][...]
        logit = jnp.sum(eh * u[:, None, :], axis=2)
        p = jax.nn.softmax(logit, axis=1)
        en = tables[hop + 1][...]
        o = jnp.sum(en * p[:, :, None], axis=1)
        u = u + o
    lg_out[...] = logit
    u_out[...] = u


def _tc_hops(E0, E1, E2, E3, hidden):
    espec = pl.BlockSpec((BB, M, D), lambda i: (i, 0, 0))
    return pl.pallas_call(
        _hops_body,
        grid=(B // BB,),
        in_specs=[espec, espec, espec, espec,
                  pl.BlockSpec((BB, 1, D), lambda i: (i, 0, 0))],
        out_specs=[pl.BlockSpec((BB, M), lambda i: (i, 0)),
                   pl.BlockSpec((BB, D), lambda i: (i, 0))],
        out_shape=[jax.ShapeDtypeStruct((B, M), jnp.float32),
                   jax.ShapeDtypeStruct((B, D), jnp.float32)],
    )(E0, E1, E2, E3, hidden)


def kernel(story, hidden, C0, C1, C2, C3):
    flat_idx = story.reshape(-1)
    E0, E1, E2, E3 = _sc_gather_pool(flat_idx, C0, C1, C2, C3)
    logit, u = _tc_hops(E0.reshape(B, M, D), E1.reshape(B, M, D),
                        E2.reshape(B, M, D), E3.reshape(B, M, D), hidden)
    return (logit, u)


# speedup vs baseline: 6.7317x; 1.8141x over previous
import functools

import jax
import jax.numpy as jnp
from jax import lax
from jax.experimental import pallas as pl
from jax.experimental.pallas import tpu as pltpu
from jax.experimental.pallas import tpu_sc as plsc

VOCAB = 100000
D = 128
B, M, S = 1024, 200, 4
POS = B * M
NC, NS = 2, 16
NW = NC * NS
POS_W = POS // NW
IPG = 128
GPP = 64
RPP = GPP * S
STEPS = POS_W // GPP
IDX_ROWS = POS_W * S // IPG


def _sc_gather_pool(idx3, C0, C1, C2, C3):
    mesh = plsc.VectorSubcoreMesh(core_axis_name="c", subcore_axis_name="s")
    out = jax.ShapeDtypeStruct((POS, D), jnp.float32)

    @functools.partial(
        pl.kernel,
        mesh=mesh,
        out_type=[out, out, out, out],
        scratch_types=[
            pltpu.VMEM((IDX_ROWS, IPG), jnp.int32),
            pltpu.VMEM((RPP, D), jnp.float32),
            pltpu.VMEM((RPP, D), jnp.float32),
            pltpu.VMEM((GPP, D), jnp.float32),
            pltpu.VMEM((GPP, D), jnp.float32),
            pltpu.SemaphoreType.DMA,
            pltpu.SemaphoreType.DMA,
            pltpu.SemaphoreType.DMA,
            pltpu.SemaphoreType.DMA,
        ],
    )
    def k(idx_hbm, t0, t1, t2, t3, e0, e1, e2, e3,
          idx_v, raw0, raw1, pool0, pool1, gsem0, gsem1, osem0, osem1):
        wid = lax.axis_index("s") * NC + lax.axis_index("c")
        base = wid * POS_W
        pltpu.sync_copy(idx_hbm.at[wid], idx_v)

        def fire_gather(table, s, raw, gsem):
            pltpu.async_copy(table.at[idx_v.at[2 * s]],
                             raw.at[pl.ds(0, IPG)], gsem)
            pltpu.async_copy(table.at[idx_v.at[2 * s + 1]],
                             raw.at[pl.ds(IPG, IPG)], gsem)

        def wait_gather(table, s, raw, gsem):
            pltpu.make_async_copy(table.at[idx_v.at[2 * s]],
                                  raw.at[pl.ds(0, IPG)], gsem).wait()
            pltpu.make_async_copy(table.at[idx_v.at[2 * s + 1]],
                                  raw.at[pl.ds(IPG, IPG)], gsem).wait()

        def fire_out(e_out, s, pool, osem):
            pltpu.async_copy(pool, e_out.at[pl.ds(base + s * GPP, GPP)], osem)

        def wait_out(e_out, s, pool, osem):
            pltpu.make_async_copy(
                pool, e_out.at[pl.ds(base + s * GPP, GPP)], osem).wait()

        def pool(raw, pool_v):
            def body(i, _):
                r = i * S
                for j in range(D // 16):
                    sl = pl.ds(j * 16, 16)
                    pool_v[i, sl] = ((raw[r, sl] + raw[r + 1, sl])
                                     + (raw[r + 2, sl] + raw[r + 3, sl]))
                return 0
            lax.fori_loop(0, GPP, body, 0)

        for table, e_out in ((t0, e0), (t1, e1), (t2, e2), (t3, e3)):
            def step_pair(su, _, table=table, e_out=e_out):
                a = 2 * su
                b = a + 1
                fire_gather(table, b, raw1, gsem1)
                wait_gather(table, a, raw0, gsem0)

                @pl.when(a >= 2)
                def _():
                    wait_out(e_out, a - 2, pool0, osem0)
                pool(raw0, pool0)
                fire_out(e_out, a, pool0, osem0)

                @pl.when(a + 2 < STEPS)
                def _():
                    fire_gather(table, a + 2, raw0, gsem0)
                wait_gather(table, b, raw1, gsem1)

                @pl.when(b >= 2)
                def _():
                    wait_out(e_out, b - 2, pool1, osem1)
                pool(raw1, pool1)
                fire_out(e_out, b, pool1, osem1)
                return 0

            fire_gather(table, 0, raw0, gsem0)
            lax.fori_loop(0, STEPS // 2, step_pair, 0)
            wait_out(e_out, STEPS - 2, pool0, osem0)
            wait_out(e_out, STEPS - 1, pool1, osem1)

    return k(idx3, C0, C1, C2, C3)


BB = 32


def _hops_body(e0, e1, e2, e3, h, lg_out, u_out):
    u = h[:, 0, :]
    tables = (e0, e1, e2, e3)
    logit = None
    for hop in range(3):
        eh = tables[hop][...]
        logit = jnp.sum(eh * u[:, None, :], axis=2)
        p = jax.nn.softmax(logit, axis=1)
        en = tables[hop + 1][...]
        o = jnp.sum(en * p[:, :, None], axis=1)
        u = u + o
    lg_out[...] = logit
    u_out[...] = u


def _tc_hops(E0, E1, E2, E3, hidden):
    espec = pl.BlockSpec((BB, M, D), lambda i: (i, 0, 0))
    return pl.pallas_call(
        _hops_body,
        grid=(B // BB,),
        in_specs=[espec, espec, espec, espec,
                  pl.BlockSpec((BB, 1, D), lambda i: (i, 0, 0))],
        out_specs=[pl.BlockSpec((BB, M), lambda i: (i, 0)),
                   pl.BlockSpec((BB, D), lambda i: (i, 0))],
        out_shape=[jax.ShapeDtypeStruct((B, M), jnp.float32),
                   jax.ShapeDtypeStruct((B, D), jnp.float32)],
    )(E0, E1, E2, E3, hidden)


def kernel(story, hidden, C0, C1, C2, C3):
    idx3 = story.reshape(NW, IDX_ROWS, IPG)
    E0, E1, E2, E3 = _sc_gather_pool(idx3, C0, C1, C2, C3)
    logit, u = _tc_hops(E0.reshape(B, M, D), E1.reshape(B, M, D),
                        E2.reshape(B, M, D), E3.reshape(B, M, D), hidden)
    return (logit, u)


# speedup vs baseline: 11.2437x; 1.6703x over previous
import functools

import jax
import jax.numpy as jnp
from jax import lax
from jax.experimental import pallas as pl
from jax.experimental.pallas import tpu as pltpu
from jax.experimental.pallas import tpu_sc as plsc

VOCAB = 100000
D = 128
B, M, S = 1024, 200, 4
POS = B * M
NC, NS = 2, 16
NW = NC * NS
POS_W = POS // NW
IPG = 128
GPP = 64
RPP = GPP * S
STEPS = POS_W // GPP
IDX_ROWS = POS_W * S // IPG


def _sc_gather_pool(idx3, C0, C1, C2, C3):
    mesh = plsc.VectorSubcoreMesh(core_axis_name="c", subcore_axis_name="s")
    out = jax.ShapeDtypeStruct((POS, D), jnp.float32)

    @functools.partial(
        pl.kernel,
        mesh=mesh,
        out_type=[out, out, out, out],
        scratch_types=[
            pltpu.VMEM((IDX_ROWS, IPG), jnp.int32),
            pltpu.VMEM((RPP, D), jnp.float32),
            pltpu.VMEM((RPP, D), jnp.float32),
            pltpu.VMEM((GPP, D), jnp.float32),
            pltpu.VMEM((GPP, D), jnp.float32),
            pltpu.SemaphoreType.DMA,
            pltpu.SemaphoreType.DMA,
            pltpu.SemaphoreType.DMA,
            pltpu.SemaphoreType.DMA,
        ],
    )
    def k(idx_hbm, t0, t1, t2, t3, e0, e1, e2, e3,
          idx_v, raw0, raw1, pool0, pool1, gsem0, gsem1, osem0, osem1):
        wid = lax.axis_index("s") * NC + lax.axis_index("c")
        base = wid * POS_W
        pltpu.sync_copy(idx_hbm.at[wid], idx_v)

        def fire_gather(table, s, raw, gsem):
            pltpu.async_copy(table.at[idx_v.at[2 * s]],
                             raw.at[pl.ds(0, IPG)], gsem)
            pltpu.async_copy(table.at[idx_v.at[2 * s + 1]],
                             raw.at[pl.ds(IPG, IPG)], gsem)

        def wait_gather(table, s, raw, gsem):
            pltpu.make_async_copy(table.at[idx_v.at[2 * s]],
                                  raw.at[pl.ds(0, IPG)], gsem).wait()
            pltpu.make_async_copy(table.at[idx_v.at[2 * s + 1]],
                                  raw.at[pl.ds(IPG, IPG)], gsem).wait()

        def fire_out(e_out, s, pool, osem):
            pltpu.async_copy(pool, e_out.at[pl.ds(base + s * GPP, GPP)], osem)

        def wait_out(e_out, s, pool, osem):
            pltpu.make_async_copy(
                pool, e_out.at[pl.ds(base + s * GPP, GPP)], osem).wait()

        def pool(raw, pool_v):
            @plsc.parallel_loop(0, GPP, unroll=4)
            def _(i):
                r = i * S
                for j in range(D // 16):
                    sl = pl.ds(j * 16, 16)
                    pool_v[i, sl] = ((raw[r, sl] + raw[r + 1, sl])
                                     + (raw[r + 2, sl] + raw[r + 3, sl]))

        for table, e_out in ((t0, e0), (t1, e1), (t2, e2), (t3, e3)):
            def step_pair(su, _, table=table, e_out=e_out):
                a = 2 * su
                b = a + 1
                fire_gather(table, b, raw1, gsem1)
                wait_gather(table, a, raw0, gsem0)

                @pl.when(a >= 2)
                def _():
                    wait_out(e_out, a - 2, pool0, osem0)
                pool(raw0, pool0)
                fire_out(e_out, a, pool0, osem0)

                @pl.when(a + 2 < STEPS)
                def _():
                    fire_gather(table, a + 2, raw0, gsem0)
                wait_gather(table, b, raw1, gsem1)

                @pl.when(b >= 2)
                def _():
                    wait_out(e_out, b - 2, pool1, osem1)
                pool(raw1, pool1)
                fire_out(e_out, b, pool1, osem1)
                return 0

            fire_gather(table, 0, raw0, gsem0)
            lax.fori_loop(0, STEPS // 2, step_pair, 0)
            wait_out(e_out, STEPS - 2, pool0, osem0)
            wait_out(e_out, STEPS - 1, pool1, osem1)

    return k(idx3, C0, C1, C2, C3)


BB = 32


def _hops_body(e0, e1, e2, e3, h, lg_out, u_out):
    u = h[:, 0, :]
    tables = (e0, e1, e2, e3)
    logit = None
    for hop in range(3):
        eh = tables[hop][...]
        logit = jnp.sum(eh * u[:, None, :], axis=2)
        p = jax.nn.softmax(logit, axis=1)
        en = tables[hop + 1][...]
        o = jnp.sum(en * p[:, :, None], axis=1)
        u = u + o
    lg_out[...] = logit
    u_out[...] = u


def _tc_hops(E0, E1, E2, E3, hidden):
    espec = pl.BlockSpec((BB, M, D), lambda i: (i, 0, 0))
    return pl.pallas_call(
        _hops_body,
        grid=(B // BB,),
        in_specs=[espec, espec, espec, espec,
                  pl.BlockSpec((BB, 1, D), lambda i: (i, 0, 0))],
        out_specs=[pl.BlockSpec((BB, M), lambda i: (i, 0)),
                   pl.BlockSpec((BB, D), lambda i: (i, 0))],
        out_shape=[jax.ShapeDtypeStruct((B, M), jnp.float32),
                   jax.ShapeDtypeStruct((B, D), jnp.float32)],
    )(E0, E1, E2, E3, hidden)


def kernel(story, hidden, C0, C1, C2, C3):
    idx3 = story.reshape(NW, IDX_ROWS, IPG)
    E0, E1, E2, E3 = _sc_gather_pool(idx3, C0, C1, C2, C3)
    logit, u = _tc_hops(E0.reshape(B, M, D), E1.reshape(B, M, D),
                        E2.reshape(B, M, D), E3.reshape(B, M, D), hidden)
    return (logit, u)


# speedup vs baseline: 11.5430x; 1.0266x over previous
import functools

import jax
import jax.numpy as jnp
from jax import lax
from jax.experimental import pallas as pl
from jax.experimental.pallas import tpu as pltpu
from jax.experimental.pallas import tpu_sc as plsc

VOCAB = 100000
D = 128
B, M, S = 1024, 200, 4
NC, NS = 2, 16
NW = NC * NS
IPG = 128
NSPLIT = 4
BS = B // NSPLIT
POS_SL = BS * M
POS_W = POS_SL // NW
GPP = 32
RPP = GPP * S
STEPS = POS_W // GPP
IDX_ROWS = POS_W * S // IPG


def _sc_gather_pool(idx3, C0, C1, C2, C3):
    mesh = plsc.VectorSubcoreMesh(core_axis_name="c", subcore_axis_name="s")
    out = jax.ShapeDtypeStruct((POS_SL, D), jnp.float32)

    @functools.partial(
        pl.kernel,
        mesh=mesh,
        out_type=[out, out, out, out],
        scratch_types=[
            pltpu.VMEM((IDX_ROWS, IPG), jnp.int32),
            pltpu.VMEM((RPP, D), jnp.float32),
            pltpu.VMEM((RPP, D), jnp.float32),
            pltpu.VMEM((GPP, D), jnp.float32),
            pltpu.VMEM((GPP, D), jnp.float32),
            pltpu.SemaphoreType.DMA,
            pltpu.SemaphoreType.DMA,
            pltpu.SemaphoreType.DMA,
            pltpu.SemaphoreType.DMA,
        ],
    )
    def k(idx_hbm, t0, t1, t2, t3, e0, e1, e2, e3,
          idx_v, raw0, raw1, pool0, pool1, gsem0, gsem1, osem0, osem1):
        wid = lax.axis_index("s") * NC + lax.axis_index("c")
        base = wid * POS_W
        pltpu.sync_copy(idx_hbm.at[wid], idx_v)

        def fire_gather(table, s, raw, gsem):
            pltpu.async_copy(table.at[idx_v.at[s]], raw, gsem)

        def wait_gather(table, s, raw, gsem):
            pltpu.make_async_copy(table.at[idx_v.at[s]], raw, gsem).wait()

        def fire_out(e_out, s, pool, osem):
            pltpu.async_copy(pool, e_out.at[pl.ds(base + s * GPP, GPP)], osem)

        def wait_out(e_out, s, pool, osem):
            pltpu.make_async_copy(
                pool, e_out.at[pl.ds(base + s * GPP, GPP)], osem).wait()

        def pool(raw, pool_v):
            @plsc.parallel_loop(0, GPP, unroll=4)
            def _(i):
                r = i * S
                for j in range(D // 16):
                    sl = pl.ds(j * 16, 16)
                    pool_v[i, sl] = ((raw[r, sl] + raw[r + 1, sl])
                                     + (raw[r + 2, sl] + raw[r + 3, sl]))

        for table, e_out in ((t0, e0), (t1, e1), (t2, e2), (t3, e3)):
            def step_pair(su, _, table=table, e_out=e_out):
                a = 2 * su
                b = a + 1
                fire_gather(table, b, raw1, gsem1)
                wait_gather(table, a, raw0, gsem0)

                @pl.when(a >= 2)
                def _():
                    wait_out(e_out, a - 2, pool0, osem0)
                pool(raw0, pool0)
                fire_out(e_out, a, pool0, osem0)

                @pl.when(a + 2 < STEPS)
                def _():
                    fire_gather(table, a + 2, raw0, gsem0)
                wait_gather(table, b, raw1, gsem1)

                @pl.when(b >= 2)
                def _():
                    wait_out(e_out, b - 2, pool1, osem1)
                pool(raw1, pool1)
                fire_out(e_out, b, pool1, osem1)
                return 0

            fire_gather(table, 0, raw0, gsem0)
            lax.fori_loop(0, STEPS // 2, step_pair, 0)
            wait_out(e_out, STEPS - 2, pool0, osem0)
            wait_out(e_out, STEPS - 1, pool1, osem1)

    return k(idx3, C0, C1, C2, C3)


BB = 32


def _hops_body(e0, e1, e2, e3, h, lg_out, u_out):
    u = h[:, 0, :]
    tables = (e0, e1, e2, e3)
    logit = None
    for hop in range(3):
        eh = tables[hop][...]
        logit = jnp.sum(eh * u[:, None, :], axis=2)
        p = jax.nn.softmax(logit, axis=1)
        en = tables[hop + 1][...]
        o = jnp.sum(en * p[:, :, None], axis=1)
        u = u + o
    lg_out[...] = logit
    u_out[...] = u


def _tc_hops(E0, E1, E2, E3, hidden):
    espec = pl.BlockSpec((BB, M, D), lambda i: (i, 0, 0))
    return pl.pallas_call(
        _hops_body,
        grid=(BS // BB,),
        in_specs=[espec, espec, espec, espec,
                  pl.BlockSpec((BB, 1, D), lambda i: (i, 0, 0))],
        out_specs=[pl.BlockSpec((BB, M), lambda i: (i, 0)),
                   pl.BlockSpec((BB, D), lambda i: (i, 0))],
        out_shape=[jax.ShapeDtypeStruct((BS, M), jnp.float32),
                   jax.ShapeDtypeStruct((BS, D), jnp.float32)],
    )(E0, E1, E2, E3, hidden)


def kernel(story, hidden, C0, C1, C2, C3):
    logits, us = [], []
    for k in range(NSPLIT):
        sl = slice(k * BS, (k + 1) * BS)
        idx3 = story[sl].reshape(NW, IDX_ROWS, IPG)
        E0, E1, E2, E3 = _sc_gather_pool(idx3, C0, C1, C2, C3)
        lg, u = _tc_hops(E0.reshape(BS, M, D), E1.reshape(BS, M, D),
                         E2.reshape(BS, M, D), E3.reshape(BS, M, D),
                         hidden[sl])
        logits.append(lg)
        us.append(u)
    return (jnp.concatenate(logits, axis=0), jnp.concatenate(us, axis=0))


# speedup vs baseline: 12.4584x; 1.0793x over previous
import functools

import jax
import jax.numpy as jnp
from jax import lax
from jax.experimental import pallas as pl
from jax.experimental.pallas import tpu as pltpu
from jax.experimental.pallas import tpu_sc as plsc

VOCAB = 100000
D = 128
B, M, S = 1024, 200, 4
NC, NS = 2, 16
NW = NC * NS
IPG = 128
NSPLIT = 4
BS = B // NSPLIT
POS_SL = BS * M
POS_W = POS_SL // NW
GPP = 64
RPP = GPP * S
STEPS = POS_W // GPP
IDX_ROWS = POS_W * S // IPG


def _sc_gather_pool(idx3, C0, C1, C2, C3):
    mesh = plsc.VectorSubcoreMesh(core_axis_name="c", subcore_axis_name="s")
    out = jax.ShapeDtypeStruct((POS_SL, D), jnp.float32)

    @functools.partial(
        pl.kernel,
        mesh=mesh,
        out_type=[out, out, out, out],
        scratch_types=[
            pltpu.VMEM((IDX_ROWS, IPG), jnp.int32),
            pltpu.VMEM((RPP, D), jnp.float32),
            pltpu.VMEM((RPP, D), jnp.float32),
            pltpu.VMEM((GPP, D), jnp.float32),
            pltpu.VMEM((GPP, D), jnp.float32),
            pltpu.SemaphoreType.DMA,
            pltpu.SemaphoreType.DMA,
            pltpu.SemaphoreType.DMA,
            pltpu.SemaphoreType.DMA,
        ],
    )
    def k(idx_hbm, t0, t1, t2, t3, e0, e1, e2, e3,
          idx_v, raw0, raw1, pool0, pool1, gsem0, gsem1, osem0, osem1):
        wid = lax.axis_index("s") * NC + lax.axis_index("c")
        base = wid * POS_W
        pltpu.sync_copy(idx_hbm.at[wid], idx_v)

        def fire_gather(table, s, raw, gsem):
            pltpu.async_copy(table.at[idx_v.at[2 * s]],
                             raw.at[pl.ds(0, IPG)], gsem)
            pltpu.async_copy(table.at[idx_v.at[2 * s + 1]],
                             raw.at[pl.ds(IPG, IPG)], gsem)

        def wait_gather(table, s, raw, gsem):
            pltpu.make_async_copy(table.at[idx_v.at[2 * s]],
                                  raw.at[pl.ds(0, IPG)], gsem).wait()
            pltpu.make_async_copy(table.at[idx_v.at[2 * s + 1]],
                                  raw.at[pl.ds(IPG, IPG)], gsem).wait()

        def fire_out(e_out, s, pool, osem):
            pltpu.async_copy(pool, e_out.at[pl.ds(base + s * GPP, GPP)], osem)

        def wait_out(e_out, s, pool, osem):
            pltpu.make_async_copy(
                pool, e_out.at[pl.ds(base + s * GPP, GPP)], osem).wait()

        def pool(raw, pool_v):
            @plsc.parallel_loop(0, GPP, unroll=4)
            def _(i):
                r = i * S
                for j in range(D // 16):
                    sl = pl.ds(j * 16, 16)
                    pool_v[i, sl] = ((raw[r, sl] + raw[r + 1, sl])
                                     + (raw[r + 2, sl] + raw[r + 3, sl]))

        for table, e_out in ((t0, e0), (t1, e1), (t2, e2), (t3, e3)):
            def step_pair(su, _, table=table, e_out=e_out):
                a = 2 * su
                b = a + 1
                fire_gather(table, b, raw1, gsem1)
                wait_gather(table, a, raw0, gsem0)

                @pl.when(a >= 2)
                def _():
                    wait_out(e_out, a - 2, pool0, osem0)
                pool(raw0, pool0)
                fire_out(e_out, a, pool0, osem0)

                @pl.when(a + 2 < STEPS)
                def _():
                    fire_gather(table, a + 2, raw0, gsem0)
                wait_gather(table, b, raw1, gsem1)

                @pl.when(b >= 2)
                def _():
                    wait_out(e_out, b - 2, pool1, osem1)
                pool(raw1, pool1)
                fire_out(e_out, b, pool1, osem1)
                return 0

            fire_gather(table, 0, raw0, gsem0)
            lax.fori_loop(0, STEPS // 2, step_pair, 0)
            if STEPS % 2 == 1:
                s_last = STEPS - 1
                wait_gather(table, s_last, raw0, gsem0)
                wait_out(e_out, s_last - 2, pool0, osem0)
                pool(raw0, pool0)
                fire_out(e_out, s_last, pool0, osem0)
                wait_out(e_out, s_last - 1, pool1, osem1)
                wait_out(e_out, s_last, pool0, osem0)
            else:
                wait_out(e_out, STEPS - 2, pool0, osem0)
                wait_out(e_out, STEPS - 1, pool1, osem1)

    return k(idx3, C0, C1, C2, C3)


BB = 32


def _hops_body(e0, e1, e2, e3, h, lg_out, u_out):
    u = h[:, 0, :]
    tables = (e0, e1, e2, e3)
    logit = None
    for hop in range(3):
        eh = tables[hop][...]
        logit = jnp.sum(eh * u[:, None, :], axis=2)
        p = jax.nn.softmax(logit, axis=1)
        en = tables[hop + 1][...]
        o = jnp.sum(en * p[:, :, None], axis=1)
        u = u + o
    lg_out[...] = logit
    u_out[...] = u


def _tc_hops(E0, E1, E2, E3, hidden):
    espec = pl.BlockSpec((BB, M, D), lambda i: (i, 0, 0))
    return pl.pallas_call(
        _hops_body,
        grid=(BS // BB,),
        in_specs=[espec, espec, espec, espec,
                  pl.BlockSpec((BB, 1, D), lambda i: (i, 0, 0))],
        out_specs=[pl.BlockSpec((BB, M), lambda i: (i, 0)),
                   pl.BlockSpec((BB, D), lambda i: (i, 0))],
        out_shape=[jax.ShapeDtypeStruct((BS, M), jnp.float32),
                   jax.ShapeDtypeStruct((BS, D), jnp.float32)],
    )(E0, E1, E2, E3, hidden)


def kernel(story, hidden, C0, C1, C2, C3):
    logits, us = [], []
    for k in range(NSPLIT):
        sl = slice(k * BS, (k + 1) * BS)
        idx3 = story[sl].reshape(NW, IDX_ROWS, IPG)
        E0, E1, E2, E3 = _sc_gather_pool(idx3, C0, C1, C2, C3)
        lg, u = _tc_hops(E0.reshape(BS, M, D), E1.reshape(BS, M, D),
                         E2.reshape(BS, M, D), E3.reshape(BS, M, D),
                         hidden[sl])
        logits.append(lg)
        us.append(u)
    return (jnp.concatenate(logits, axis=0), jnp.concatenate(us, axis=0))
